# initial kernel scaffold (unmeasured)
import jax
import jax.numpy as jnp
from jax import lax
from jax.experimental import pallas as pl
from jax.experimental.pallas import tpu as pltpu

N_Z = 4
E_LOCAL = 4
N_E = N_Z * E_LOCAL
CAP = 160


def _body(xs_ref, w1_hbm, w2_hbm, res_ref, recv_ref, w1buf, w2buf,
          loc_sems, fwd_send, fwd_recv, bwd_send, bwd_recv, wsem1, wsem2):
    my_x = lax.axis_index("x")
    my_y = lax.axis_index("y")
    my_z = lax.axis_index("z")

    barrier_sem = pltpu.get_barrier_semaphore()
    for dz in range(1, N_Z):
        pl.semaphore_signal(
            barrier_sem, inc=1,
            device_id=(my_x, my_y, (my_z + dz) % N_Z),
            device_id_type=pl.DeviceIdType.MESH,
        )
    pl.semaphore_wait(barrier_sem, N_Z - 1)

    loc_fwd = []
    for j in range(E_LOCAL):
        cp = pltpu.make_async_copy(
            xs_ref.at[pl.ds(E_LOCAL * my_z + j, 1)],
            recv_ref.at[j, pl.ds(0, 1)],
            loc_sems.at[j],
        )
        cp.start()
        loc_fwd.append(cp)

    fwd = []
    for dz in range(1, N_Z):
        d = (my_z + dz) % N_Z
        for j in range(E_LOCAL):
            rdma = pltpu.make_async_remote_copy(
                src_ref=xs_ref.at[pl.ds(E_LOCAL * d + j, 1)],
                dst_ref=recv_ref.at[j, pl.ds(dz, 1)],
                send_sem=fwd_send.at[j, dz],
                recv_sem=fwd_recv.at[j, dz],
                device_id=(my_x, my_y, d),
                device_id_type=pl.DeviceIdType.MESH,
            )
            rdma.start()
            fwd.append(rdma)

    for cp in loc_fwd:
        cp.wait()
    for b in range(1, N_Z):
        for j in range(E_LOCAL):
            rcv = pltpu.make_async_remote_copy(
                src_ref=xs_ref.at[pl.ds(0, 1)],
                dst_ref=recv_ref.at[j, pl.ds(b, 1)],
                send_sem=fwd_send.at[j, b],
                recv_sem=fwd_recv.at[j, b],
                device_id=(my_x, my_y, my_z),
                device_id_type=pl.DeviceIdType.MESH,
            )
            rcv.wait_recv()
    for rdma in fwd:
        rdma.wait_send()

    for j in range(E_LOCAL):
        cp1 = pltpu.make_async_copy(w1_hbm.at[j], w1buf, wsem1)
        cp2 = pltpu.make_async_copy(w2_hbm.at[j], w2buf, wsem2)
        cp1.start()
        cp2.start()
        cp1.wait()
        a = recv_ref[j].reshape(N_Z * CAP, xs_ref.shape[2])
        h = jnp.maximum(
            jax.lax.dot(a, w1buf[...], preferred_element_type=jnp.float32), 0.0
        )
        cp2.wait()
        o = jax.lax.dot(h, w2buf[...], preferred_element_type=jnp.float32)
        recv_ref[j] = o.reshape(N_Z, CAP, xs_ref.shape[2])

    loc_bwd = []
    for j in range(E_LOCAL):
        cp = pltpu.make_async_copy(
            recv_ref.at[j, pl.ds(0, 1)],
            res_ref.at[pl.ds(E_LOCAL * my_z + j, 1)],
            loc_sems.at[j],
        )
        cp.start()
        loc_bwd.append(cp)

    bwd = []
    for b in range(1, N_Z):
        s = (my_z - b) % N_Z
        for j in range(E_LOCAL):
            rdma = pltpu.make_async_remote_copy(
                src_ref=recv_ref.at[j, pl.ds(b, 1)],
                dst_ref=res_ref.at[pl.ds(E_LOCAL * my_z + j, 1)],
                send_sem=bwd_send.at[j, b],
                recv_sem=bwd_recv.at[j, b],
                device_id=(my_x, my_y, s),
                device_id_type=pl.DeviceIdType.MESH,
            )
            rdma.start()
            bwd.append(rdma)

    for cp in loc_bwd:
        cp.wait()
    for f in range(1, N_Z):
        for j in range(E_LOCAL):
            e = E_LOCAL * ((my_z + f) % N_Z) + j
            rcv = pltpu.make_async_remote_copy(
                src_ref=recv_ref.at[j, pl.ds(0, 1)],
                dst_ref=res_ref.at[pl.ds(e, 1)],
                send_sem=bwd_send.at[j, f],
                recv_sem=bwd_recv.at[j, f],
                device_id=(my_x, my_y, my_z),
                device_id_type=pl.DeviceIdType.MESH,
            )
            rcv.wait_recv()
    for rdma in bwd:
        rdma.wait_send()


def kernel(x, assign, W1, W2):
    T, D = x.shape

    counts = jnp.bincount(assign, length=N_E)
    order = jnp.argsort(assign)
    starts = jnp.concatenate(
        [jnp.zeros((1,), counts.dtype), jnp.cumsum(counts)[:-1]]
    )
    c = jnp.arange(CAP, dtype=jnp.int32)
    pos = starts[:, None] + c[None, :]
    valid = c[None, :] < counts[:, None]
    idx = jnp.where(valid, order[jnp.clip(pos, 0, T - 1)], T)
    xs = x[jnp.clip(idx, 0, T - 1)]

    res = pl.pallas_call(
        _body,
        out_shape=jax.ShapeDtypeStruct((N_E, CAP, D), jnp.float32),
        in_specs=[
            pl.BlockSpec(memory_space=pltpu.VMEM),
            pl.BlockSpec(memory_space=pltpu.ANY),
            pl.BlockSpec(memory_space=pltpu.ANY),
        ],
        out_specs=pl.BlockSpec(memory_space=pltpu.VMEM),
        scratch_shapes=[
            pltpu.VMEM((E_LOCAL, N_Z, CAP, D), jnp.float32),
            pltpu.VMEM(W1.shape[1:], jnp.float32),
            pltpu.VMEM(W2.shape[1:], jnp.float32),
            pltpu.SemaphoreType.DMA((E_LOCAL,)),
            pltpu.SemaphoreType.DMA((E_LOCAL, N_Z)),
            pltpu.SemaphoreType.DMA((E_LOCAL, N_Z)),
            pltpu.SemaphoreType.DMA((E_LOCAL, N_Z)),
            pltpu.SemaphoreType.DMA((E_LOCAL, N_Z)),
            pltpu.SemaphoreType.DMA,
            pltpu.SemaphoreType.DMA,
        ],
        compiler_params=pltpu.CompilerParams(collective_id=0),
    )(xs, W1, W2)

    out = jnp.zeros_like(x).at[idx.reshape(-1)].set(
        res.reshape(-1, D), mode="drop"
    )
    return out


# baseline (device time: 893158 ns/iter reference)
import jax
import jax.numpy as jnp
from jax import lax
from jax.experimental import pallas as pl
from jax.experimental.pallas import tpu as pltpu

N_Z = 4
E_LOCAL = 4
N_E = N_Z * E_LOCAL
CAP = 160


def _body(xs_ref, w1_hbm, w2_hbm, res_ref, recv_ref, w1buf, w2buf,
          loc_sems, fwd_send, fwd_recv, bwd_send, bwd_recv, wsem1, wsem2):
    my_x = lax.axis_index("x")
    my_y = lax.axis_index("y")
    my_z = lax.axis_index("z")

    barrier_sem = pltpu.get_barrier_semaphore()
    for dz in range(1, N_Z):
        pl.semaphore_signal(
            barrier_sem, inc=1,
            device_id=(my_x, my_y, (my_z + dz) % N_Z),
            device_id_type=pl.DeviceIdType.MESH,
        )
    pl.semaphore_wait(barrier_sem, N_Z - 1)

    loc_fwd = []
    for j in range(E_LOCAL):
        cp = pltpu.make_async_copy(
            xs_ref.at[pl.ds(E_LOCAL * my_z + j, 1)],
            recv_ref.at[j, pl.ds(0, 1)],
            loc_sems.at[j],
        )
        cp.start()
        loc_fwd.append(cp)

    fwd = []
    for dz in range(1, N_Z):
        d = (my_z + dz) % N_Z
        for j in range(E_LOCAL):
            rdma = pltpu.make_async_remote_copy(
                src_ref=xs_ref.at[pl.ds(E_LOCAL * d + j, 1)],
                dst_ref=recv_ref.at[j, pl.ds(dz, 1)],
                send_sem=fwd_send.at[j, dz],
                recv_sem=fwd_recv.at[j, dz],
                device_id=(my_x, my_y, d),
                device_id_type=pl.DeviceIdType.MESH,
            )
            rdma.start()
            fwd.append(rdma)

    for cp in loc_fwd:
        cp.wait()
    for b in range(1, N_Z):
        for j in range(E_LOCAL):
            rcv = pltpu.make_async_remote_copy(
                src_ref=xs_ref.at[pl.ds(0, 1)],
                dst_ref=recv_ref.at[j, pl.ds(b, 1)],
                send_sem=fwd_send.at[j, b],
                recv_sem=fwd_recv.at[j, b],
                device_id=(my_x, my_y, my_z),
                device_id_type=pl.DeviceIdType.MESH,
            )
            rcv.wait_recv()
    for rdma in fwd:
        rdma.wait_send()

    for j in range(E_LOCAL):
        cp1 = pltpu.make_async_copy(w1_hbm.at[j], w1buf, wsem1)
        cp2 = pltpu.make_async_copy(w2_hbm.at[j], w2buf, wsem2)
        cp1.start()
        cp2.start()
        cp1.wait()
        a = recv_ref[j].reshape(N_Z * CAP, xs_ref.shape[2])
        h = jnp.maximum(
            jax.lax.dot(a, w1buf[...], preferred_element_type=jnp.float32), 0.0
        )
        cp2.wait()
        o = jax.lax.dot(h, w2buf[...], preferred_element_type=jnp.float32)
        recv_ref[j] = o.reshape(N_Z, CAP, xs_ref.shape[2])

    loc_bwd = []
    for j in range(E_LOCAL):
        cp = pltpu.make_async_copy(
            recv_ref.at[j, pl.ds(0, 1)],
            res_ref.at[pl.ds(E_LOCAL * my_z + j, 1)],
            loc_sems.at[j],
        )
        cp.start()
        loc_bwd.append(cp)

    bwd = []
    for b in range(1, N_Z):
        s = (my_z - b) % N_Z
        for j in range(E_LOCAL):
            rdma = pltpu.make_async_remote_copy(
                src_ref=recv_ref.at[j, pl.ds(b, 1)],
                dst_ref=res_ref.at[pl.ds(E_LOCAL * my_z + j, 1)],
                send_sem=bwd_send.at[j, b],
                recv_sem=bwd_recv.at[j, b],
                device_id=(my_x, my_y, s),
                device_id_type=pl.DeviceIdType.MESH,
            )
            rdma.start()
            bwd.append(rdma)

    for cp in loc_bwd:
        cp.wait()
    for f in range(1, N_Z):
        for j in range(E_LOCAL):
            e = E_LOCAL * ((my_z + f) % N_Z) + j
            rcv = pltpu.make_async_remote_copy(
                src_ref=recv_ref.at[j, pl.ds(0, 1)],
                dst_ref=res_ref.at[pl.ds(e, 1)],
                send_sem=bwd_send.at[j, f],
                recv_sem=bwd_recv.at[j, f],
                device_id=(my_x, my_y, my_z),
                device_id_type=pl.DeviceIdType.MESH,
            )
            rcv.wait_recv()
    for rdma in bwd:
        rdma.wait_send()


def kernel(x, assign, W1, W2):
    T, D = x.shape

    counts = jnp.bincount(assign, length=N_E)
    order = jnp.argsort(assign)
    starts = jnp.concatenate(
        [jnp.zeros((1,), counts.dtype), jnp.cumsum(counts)[:-1]]
    )
    c = jnp.arange(CAP, dtype=jnp.int32)
    pos = starts[:, None] + c[None, :]
    valid = c[None, :] < counts[:, None]
    idx = jnp.where(valid, order[jnp.clip(pos, 0, T - 1)], T)
    xs = x[jnp.clip(idx, 0, T - 1)]

    res = pl.pallas_call(
        _body,
        out_shape=jax.ShapeDtypeStruct((N_E, CAP, D), jnp.float32),
        in_specs=[
            pl.BlockSpec(memory_space=pltpu.VMEM),
            pl.BlockSpec(memory_space=pl.ANY),
            pl.BlockSpec(memory_space=pl.ANY),
        ],
        out_specs=pl.BlockSpec(memory_space=pltpu.VMEM),
        scratch_shapes=[
            pltpu.VMEM((E_LOCAL, N_Z, CAP, D), jnp.float32),
            pltpu.VMEM(W1.shape[1:], jnp.float32),
            pltpu.VMEM(W2.shape[1:], jnp.float32),
            pltpu.SemaphoreType.DMA((E_LOCAL,)),
            pltpu.SemaphoreType.DMA((E_LOCAL, N_Z)),
            pltpu.SemaphoreType.DMA((E_LOCAL, N_Z)),
            pltpu.SemaphoreType.DMA((E_LOCAL, N_Z)),
            pltpu.SemaphoreType.DMA((E_LOCAL, N_Z)),
            pltpu.SemaphoreType.DMA,
            pltpu.SemaphoreType.DMA,
        ],
        compiler_params=pltpu.CompilerParams(collective_id=0),
    )(xs, W1, W2)

    out = jnp.zeros_like(x).at[idx.reshape(-1)].set(
        res.reshape(-1, D), mode="drop"
    )
    return out


# device time: 319937 ns/iter; 2.7917x vs baseline; 2.7917x over previous
import jax
import jax.numpy as jnp
from jax import lax
from jax.experimental import pallas as pl
from jax.experimental.pallas import tpu as pltpu

N_Z = 4
E_LOCAL = 4
N_E = N_Z * E_LOCAL
CAP = 160
CHUNK = N_Z * CAP


def _body(x_ref, p_ref, w1_hbm, w2_hbm, out_ref, xs_ref, recv_ref,
          w1buf, w2buf, loc_sems, fwd_send, fwd_recv, bwd_send, bwd_recv,
          wsem1, wsem2):
    my_x = lax.axis_index("x")
    my_y = lax.axis_index("y")
    my_z = lax.axis_index("z")
    T, D = x_ref.shape

    barrier_sem = pltpu.get_barrier_semaphore()
    for dz in range(1, N_Z):
        pl.semaphore_signal(
            barrier_sem, inc=1,
            device_id=(my_x, my_y, (my_z + dz) % N_Z),
            device_id_type=pl.DeviceIdType.MESH,
        )
    pl.semaphore_wait(barrier_sem, N_Z - 1)

    p_row = p_ref[...].reshape(1, T)
    xv = x_ref[...]
    for k in range(N_E // E_LOCAL):
        q = jax.lax.broadcasted_iota(jnp.int32, (CHUNK, T), 0) + k * CHUNK
        ohT = (q == p_row).astype(jnp.float32)
        chunk = jax.lax.dot(ohT, xv, preferred_element_type=jnp.float32)
        xs_ref[pl.ds(E_LOCAL * k, E_LOCAL)] = chunk.reshape(E_LOCAL, CAP, D)

    loc_fwd = []
    for j in range(E_LOCAL):
        cp = pltpu.make_async_copy(
            xs_ref.at[pl.ds(E_LOCAL * my_z + j, 1)],
            recv_ref.at[j, pl.ds(0, 1)],
            loc_sems.at[j],
        )
        cp.start()
        loc_fwd.append(cp)

    fwd = []
    for dz in range(1, N_Z):
        d = (my_z + dz) % N_Z
        for j in range(E_LOCAL):
            rdma = pltpu.make_async_remote_copy(
                src_ref=xs_ref.at[pl.ds(E_LOCAL * d + j, 1)],
                dst_ref=recv_ref.at[j, pl.ds(dz, 1)],
                send_sem=fwd_send.at[j, dz],
                recv_sem=fwd_recv.at[j, dz],
                device_id=(my_x, my_y, d),
                device_id_type=pl.DeviceIdType.MESH,
            )
            rdma.start()
            fwd.append(rdma)

    for cp in loc_fwd:
        cp.wait()
    for b in range(1, N_Z):
        for j in range(E_LOCAL):
            rcv = pltpu.make_async_remote_copy(
                src_ref=xs_ref.at[pl.ds(0, 1)],
                dst_ref=recv_ref.at[j, pl.ds(b, 1)],
                send_sem=fwd_send.at[j, b],
                recv_sem=fwd_recv.at[j, b],
                device_id=(my_x, my_y, my_z),
                device_id_type=pl.DeviceIdType.MESH,
            )
            rcv.wait_recv()
    for rdma in fwd:
        rdma.wait_send()

    for j in range(E_LOCAL):
        cp1 = pltpu.make_async_copy(w1_hbm.at[j], w1buf, wsem1)
        cp2 = pltpu.make_async_copy(w2_hbm.at[j], w2buf, wsem2)
        cp1.start()
        cp2.start()
        cp1.wait()
        a = recv_ref[j].reshape(N_Z * CAP, D)
        h = jnp.maximum(
            jax.lax.dot(a, w1buf[...], preferred_element_type=jnp.float32), 0.0
        )
        cp2.wait()
        o = jax.lax.dot(h, w2buf[...], preferred_element_type=jnp.float32)
        recv_ref[j] = o.reshape(N_Z, CAP, D)

    loc_bwd = []
    for j in range(E_LOCAL):
        cp = pltpu.make_async_copy(
            recv_ref.at[j, pl.ds(0, 1)],
            xs_ref.at[pl.ds(E_LOCAL * my_z + j, 1)],
            loc_sems.at[j],
        )
        cp.start()
        loc_bwd.append(cp)

    bwd = []
    for b in range(1, N_Z):
        s = (my_z - b) % N_Z
        for j in range(E_LOCAL):
            rdma = pltpu.make_async_remote_copy(
                src_ref=recv_ref.at[j, pl.ds(b, 1)],
                dst_ref=xs_ref.at[pl.ds(E_LOCAL * my_z + j, 1)],
                send_sem=bwd_send.at[j, b],
                recv_sem=bwd_recv.at[j, b],
                device_id=(my_x, my_y, s),
                device_id_type=pl.DeviceIdType.MESH,
            )
            rdma.start()
            bwd.append(rdma)

    for cp in loc_bwd:
        cp.wait()
    for f in range(1, N_Z):
        for j in range(E_LOCAL):
            e = E_LOCAL * ((my_z + f) % N_Z) + j
            rcv = pltpu.make_async_remote_copy(
                src_ref=recv_ref.at[j, pl.ds(0, 1)],
                dst_ref=xs_ref.at[pl.ds(e, 1)],
                send_sem=bwd_send.at[j, f],
                recv_sem=bwd_recv.at[j, f],
                device_id=(my_x, my_y, my_z),
                device_id_type=pl.DeviceIdType.MESH,
            )
            rcv.wait_recv()
    for rdma in bwd:
        rdma.wait_send()

    p_col = p_ref[...].reshape(T, 1)
    acc = jnp.zeros((T, D), jnp.float32)
    for k in range(N_E // E_LOCAL):
        q = jax.lax.broadcasted_iota(jnp.int32, (T, CHUNK), 1) + k * CHUNK
        oh = (q == p_col).astype(jnp.float32)
        res_chunk = xs_ref[pl.ds(E_LOCAL * k, E_LOCAL)].reshape(CHUNK, D)
        acc = acc + jax.lax.dot(oh, res_chunk, preferred_element_type=jnp.float32)
    out_ref[...] = acc


def kernel(x, assign, W1, W2):
    T, D = x.shape

    onehot = assign[:, None] == jnp.arange(N_E, dtype=assign.dtype)[None, :]
    cum = jnp.cumsum(onehot.astype(jnp.int32), axis=0)
    rank = jnp.sum(jnp.where(onehot, cum - 1, 0), axis=1)
    p = assign * CAP + rank

    return pl.pallas_call(
        _body,
        out_shape=jax.ShapeDtypeStruct((T, D), jnp.float32),
        in_specs=[
            pl.BlockSpec(memory_space=pltpu.VMEM),
            pl.BlockSpec(memory_space=pltpu.VMEM),
            pl.BlockSpec(memory_space=pl.ANY),
            pl.BlockSpec(memory_space=pl.ANY),
        ],
        out_specs=pl.BlockSpec(memory_space=pltpu.VMEM),
        scratch_shapes=[
            pltpu.VMEM((N_E, CAP, D), jnp.float32),
            pltpu.VMEM((E_LOCAL, N_Z, CAP, D), jnp.float32),
            pltpu.VMEM(W1.shape[1:], jnp.float32),
            pltpu.VMEM(W2.shape[1:], jnp.float32),
            pltpu.SemaphoreType.DMA((E_LOCAL,)),
            pltpu.SemaphoreType.DMA((E_LOCAL, N_Z)),
            pltpu.SemaphoreType.DMA((E_LOCAL, N_Z)),
            pltpu.SemaphoreType.DMA((E_LOCAL, N_Z)),
            pltpu.SemaphoreType.DMA((E_LOCAL, N_Z)),
            pltpu.SemaphoreType.DMA,
            pltpu.SemaphoreType.DMA,
        ],
        compiler_params=pltpu.CompilerParams(
            collective_id=0, vmem_limit_bytes=100 * 1024 * 1024
        ),
    )(x, p.astype(jnp.int32), W1, W2)


# device time: 185516 ns/iter; 4.8145x vs baseline; 1.7246x over previous
import jax
import jax.numpy as jnp
from jax import lax
from jax.experimental import pallas as pl
from jax.experimental.pallas import tpu as pltpu

N_Z = 4
E_LOCAL = 4
N_E = N_Z * E_LOCAL
CAP = 160
CHUNK = E_LOCAL * CAP


def _body(x_ref, p_ref, w1_hbm, w2_hbm, out_ref, xs_ref, recv_ref,
          w1buf, w2buf, loc_sems, fwd_send, fwd_recv, bwd_send, bwd_recv,
          wsem1, wsem2):
    my_x = lax.axis_index("x")
    my_y = lax.axis_index("y")
    my_z = lax.axis_index("z")
    T, D = x_ref.shape

    barrier_sem = pltpu.get_barrier_semaphore()
    for dz in range(1, N_Z):
        pl.semaphore_signal(
            barrier_sem, inc=1,
            device_id=(my_x, my_y, (my_z + dz) % N_Z),
            device_id_type=pl.DeviceIdType.MESH,
        )
    pl.semaphore_wait(barrier_sem, N_Z - 1)

    p_row = p_ref[...].reshape(1, T)
    xv = x_ref[...].astype(jnp.bfloat16)

    def gather_chunk(d):
        q = jax.lax.broadcasted_iota(jnp.int32, (CHUNK, T), 0) + d * CHUNK
        ohT = (q == p_row).astype(jnp.bfloat16)
        chunk = jax.lax.dot(ohT, xv, preferred_element_type=jnp.float32)
        xs_ref[pl.ds(E_LOCAL * d, E_LOCAL)] = (
            chunk.astype(jnp.bfloat16).reshape(E_LOCAL, CAP, D)
        )

    fwd = []
    for dz in range(1, N_Z):
        d = (my_z + dz) % N_Z
        gather_chunk(d)
        for j in range(E_LOCAL):
            rdma = pltpu.make_async_remote_copy(
                src_ref=xs_ref.at[pl.ds(E_LOCAL * d + j, 1)],
                dst_ref=recv_ref.at[j, pl.ds(dz, 1)],
                send_sem=fwd_send.at[j, dz],
                recv_sem=fwd_recv.at[j, dz],
                device_id=(my_x, my_y, d),
                device_id_type=pl.DeviceIdType.MESH,
            )
            rdma.start()
            fwd.append(rdma)
    gather_chunk(my_z)
    loc_fwd = []
    for j in range(E_LOCAL):
        cp = pltpu.make_async_copy(
            xs_ref.at[pl.ds(E_LOCAL * my_z + j, 1)],
            recv_ref.at[j, pl.ds(0, 1)],
            loc_sems.at[j],
        )
        cp.start()
        loc_fwd.append(cp)

    def start_wload(j):
        buf = j % 2
        cp1 = pltpu.make_async_copy(w1_hbm.at[j], w1buf.at[buf], wsem1.at[buf])
        cp2 = pltpu.make_async_copy(w2_hbm.at[j], w2buf.at[buf], wsem2.at[buf])
        cp1.start()
        cp2.start()
        return cp1, cp2

    wload = start_wload(0)
    bwd = []
    for j in range(E_LOCAL):
        nxt = start_wload(j + 1) if j + 1 < E_LOCAL else None
        loc_fwd[j].wait()
        for b in range(1, N_Z):
            rcv = pltpu.make_async_remote_copy(
                src_ref=xs_ref.at[pl.ds(0, 1)],
                dst_ref=recv_ref.at[j, pl.ds(b, 1)],
                send_sem=fwd_send.at[j, b],
                recv_sem=fwd_recv.at[j, b],
                device_id=(my_x, my_y, my_z),
                device_id_type=pl.DeviceIdType.MESH,
            )
            rcv.wait_recv()
        wload[0].wait()
        wload[1].wait()
        buf = j % 2
        a = recv_ref[j].reshape(N_Z * CAP, D)
        h = jnp.maximum(
            jax.lax.dot(a, w1buf[buf], preferred_element_type=jnp.float32), 0.0
        ).astype(jnp.bfloat16)
        o = jax.lax.dot(h, w2buf[buf], preferred_element_type=jnp.float32)
        recv_ref[j] = o.astype(jnp.bfloat16).reshape(N_Z, CAP, D)
        wload = nxt

        cp = pltpu.make_async_copy(
            recv_ref.at[j, pl.ds(0, 1)],
            xs_ref.at[pl.ds(E_LOCAL * my_z + j, 1)],
            loc_sems.at[j],
        )
        cp.start()
        loc_fwd[j] = cp
        for b in range(1, N_Z):
            s = (my_z - b) % N_Z
            rdma = pltpu.make_async_remote_copy(
                src_ref=recv_ref.at[j, pl.ds(b, 1)],
                dst_ref=xs_ref.at[pl.ds(E_LOCAL * my_z + j, 1)],
                send_sem=bwd_send.at[j, b],
                recv_sem=bwd_recv.at[j, b],
                device_id=(my_x, my_y, s),
                device_id_type=pl.DeviceIdType.MESH,
            )
            rdma.start()
            bwd.append(rdma)

    p_col = p_ref[...].reshape(T, 1)

    def scatter_chunk(d, first):
        q = jax.lax.broadcasted_iota(jnp.int32, (T, CHUNK), 1) + d * CHUNK
        oh = (q == p_col).astype(jnp.bfloat16)
        res = xs_ref[pl.ds(E_LOCAL * d, E_LOCAL)].reshape(CHUNK, D)
        contrib = jax.lax.dot(oh, res, preferred_element_type=jnp.float32)
        if first:
            out_ref[...] = contrib
        else:
            out_ref[...] += contrib

    for j in range(E_LOCAL):
        loc_fwd[j].wait()
    scatter_chunk(my_z, True)
    for f in range(1, N_Z):
        d = (my_z + f) % N_Z
        for j in range(E_LOCAL):
            rcv = pltpu.make_async_remote_copy(
                src_ref=recv_ref.at[j, pl.ds(0, 1)],
                dst_ref=xs_ref.at[pl.ds(E_LOCAL * d + j, 1)],
                send_sem=bwd_send.at[j, f],
                recv_sem=bwd_recv.at[j, f],
                device_id=(my_x, my_y, my_z),
                device_id_type=pl.DeviceIdType.MESH,
            )
            rcv.wait_recv()
        scatter_chunk(d, False)

    for rdma in fwd:
        rdma.wait_send()
    for rdma in bwd:
        rdma.wait_send()


def kernel(x, assign, W1, W2):
    T, D = x.shape

    onehot = assign[:, None] == jnp.arange(N_E, dtype=assign.dtype)[None, :]
    cum = jnp.cumsum(onehot.astype(jnp.int32), axis=0)
    rank = jnp.sum(jnp.where(onehot, cum - 1, 0), axis=1)
    p = assign * CAP + rank

    return pl.pallas_call(
        _body,
        out_shape=jax.ShapeDtypeStruct((T, D), jnp.float32),
        in_specs=[
            pl.BlockSpec(memory_space=pltpu.VMEM),
            pl.BlockSpec(memory_space=pltpu.VMEM),
            pl.BlockSpec(memory_space=pl.ANY),
            pl.BlockSpec(memory_space=pl.ANY),
        ],
        out_specs=pl.BlockSpec(memory_space=pltpu.VMEM),
        scratch_shapes=[
            pltpu.VMEM((N_E, CAP, D), jnp.bfloat16),
            pltpu.VMEM((E_LOCAL, N_Z, CAP, D), jnp.bfloat16),
            pltpu.VMEM((2,) + W1.shape[1:], jnp.bfloat16),
            pltpu.VMEM((2,) + W2.shape[1:], jnp.bfloat16),
            pltpu.SemaphoreType.DMA((E_LOCAL,)),
            pltpu.SemaphoreType.DMA((E_LOCAL, N_Z)),
            pltpu.SemaphoreType.DMA((E_LOCAL, N_Z)),
            pltpu.SemaphoreType.DMA((E_LOCAL, N_Z)),
            pltpu.SemaphoreType.DMA((E_LOCAL, N_Z)),
            pltpu.SemaphoreType.DMA((2,)),
            pltpu.SemaphoreType.DMA((2,)),
        ],
        compiler_params=pltpu.CompilerParams(
            collective_id=0, vmem_limit_bytes=100 * 1024 * 1024
        ),
    )(x, p.astype(jnp.int32),
      W1.astype(jnp.bfloat16), W2.astype(jnp.bfloat16))


# device time: 151321 ns/iter; 5.9024x vs baseline; 1.2260x over previous
import jax
import jax.numpy as jnp
from jax import lax
from jax.experimental import pallas as pl
from jax.experimental.pallas import tpu as pltpu

N_Z = 4
E_LOCAL = 4
N_E = N_Z * E_LOCAL
CAP = 160
CHUNK = E_LOCAL * CAP


def _body(x_ref, p_ref, w1_hbm, w2_hbm, out_ref, xs_ref, recv_ref,
          w1buf, w2buf, loc_sems, fwd_send, fwd_recv, bwd_send, bwd_recv,
          wsem1, wsem2):
    my_x = lax.axis_index("x")
    my_y = lax.axis_index("y")
    my_z = lax.axis_index("z")
    T, D = x_ref.shape

    barrier_sem = pltpu.get_barrier_semaphore()
    for dz in range(1, N_Z):
        pl.semaphore_signal(
            barrier_sem, inc=1,
            device_id=(my_x, my_y, (my_z + dz) % N_Z),
            device_id_type=pl.DeviceIdType.MESH,
        )
    pl.semaphore_wait(barrier_sem, N_Z - 1)

    p_row = p_ref[...].reshape(1, T)
    xv = x_ref[...].astype(jnp.bfloat16)

    def gather_chunk(d):
        q = jax.lax.broadcasted_iota(jnp.int32, (CHUNK, T), 0) + d * CHUNK
        ohT = (q == p_row).astype(jnp.bfloat16)
        chunk = jax.lax.dot(ohT, xv, preferred_element_type=jnp.float32)
        xs_ref[pl.ds(E_LOCAL * d, E_LOCAL)] = (
            chunk.astype(jnp.bfloat16).reshape(E_LOCAL, CAP, D)
        )

    fwd = []
    for dz in range(1, N_Z):
        d = (my_z + dz) % N_Z
        gather_chunk(d)
        for j in range(E_LOCAL):
            rdma = pltpu.make_async_remote_copy(
                src_ref=xs_ref.at[pl.ds(E_LOCAL * d + j, 1)],
                dst_ref=recv_ref.at[j, pl.ds(dz, 1)],
                send_sem=fwd_send.at[j, dz],
                recv_sem=fwd_recv.at[j, dz],
                device_id=(my_x, my_y, d),
                device_id_type=pl.DeviceIdType.MESH,
            )
            rdma.start()
            fwd.append(rdma)
    gather_chunk(my_z)
    loc_fwd = []
    for j in range(E_LOCAL):
        cp = pltpu.make_async_copy(
            xs_ref.at[pl.ds(E_LOCAL * my_z + j, 1)],
            recv_ref.at[j, pl.ds(0, 1)],
            loc_sems.at[j],
        )
        cp.start()
        loc_fwd.append(cp)

    def load_w(hbm, j, buf, sem):
        cp = pltpu.make_async_copy(hbm.at[j], buf, sem)
        cp.start()
        return cp

    w1load = load_w(w1_hbm, 0, w1buf, wsem1)
    w2load = load_w(w2_hbm, 0, w2buf, wsem2)
    bwd = []
    for j in range(E_LOCAL):
        loc_fwd[j].wait()
        for b in range(1, N_Z):
            rcv = pltpu.make_async_remote_copy(
                src_ref=xs_ref.at[pl.ds(0, 1)],
                dst_ref=recv_ref.at[j, pl.ds(b, 1)],
                send_sem=fwd_send.at[j, b],
                recv_sem=fwd_recv.at[j, b],
                device_id=(my_x, my_y, my_z),
                device_id_type=pl.DeviceIdType.MESH,
            )
            rcv.wait_recv()
        a = recv_ref[j].reshape(N_Z * CAP, D).astype(jnp.float32)
        w1load.wait()
        h = jnp.maximum(
            jax.lax.dot(a, w1buf[...], preferred_element_type=jnp.float32), 0.0
        )
        if j + 1 < E_LOCAL:
            w1load = load_w(w1_hbm, j + 1, w1buf, wsem1)
        w2load.wait()
        o = jax.lax.dot(h, w2buf[...], preferred_element_type=jnp.float32)
        if j + 1 < E_LOCAL:
            w2load = load_w(w2_hbm, j + 1, w2buf, wsem2)
        recv_ref[j] = o.astype(jnp.bfloat16).reshape(N_Z, CAP, D)

        cp = pltpu.make_async_copy(
            recv_ref.at[j, pl.ds(0, 1)],
            xs_ref.at[pl.ds(E_LOCAL * my_z + j, 1)],
            loc_sems.at[j],
        )
        cp.start()
        loc_fwd[j] = cp
        for b in range(1, N_Z):
            s = (my_z - b) % N_Z
            rdma = pltpu.make_async_remote_copy(
                src_ref=recv_ref.at[j, pl.ds(b, 1)],
                dst_ref=xs_ref.at[pl.ds(E_LOCAL * my_z + j, 1)],
                send_sem=bwd_send.at[j, b],
                recv_sem=bwd_recv.at[j, b],
                device_id=(my_x, my_y, s),
                device_id_type=pl.DeviceIdType.MESH,
            )
            rdma.start()
            bwd.append(rdma)

    p_col = p_ref[...].reshape(T, 1)

    def scatter_chunk(d, first):
        q = jax.lax.broadcasted_iota(jnp.int32, (T, CHUNK), 1) + d * CHUNK
        oh = (q == p_col).astype(jnp.bfloat16)
        res = xs_ref[pl.ds(E_LOCAL * d, E_LOCAL)].reshape(CHUNK, D)
        contrib = jax.lax.dot(oh, res, preferred_element_type=jnp.float32)
        if first:
            out_ref[...] = contrib
        else:
            out_ref[...] += contrib

    for j in range(E_LOCAL):
        loc_fwd[j].wait()
    scatter_chunk(my_z, True)
    for f in range(1, N_Z):
        d = (my_z + f) % N_Z
        for j in range(E_LOCAL):
            rcv = pltpu.make_async_remote_copy(
                src_ref=recv_ref.at[j, pl.ds(0, 1)],
                dst_ref=xs_ref.at[pl.ds(E_LOCAL * d + j, 1)],
                send_sem=bwd_send.at[j, f],
                recv_sem=bwd_recv.at[j, f],
                device_id=(my_x, my_y, my_z),
                device_id_type=pl.DeviceIdType.MESH,
            )
            rcv.wait_recv()
        scatter_chunk(d, False)

    for rdma in fwd:
        rdma.wait_send()
    for rdma in bwd:
        rdma.wait_send()


def kernel(x, assign, W1, W2):
    T, D = x.shape

    onehot = assign[:, None] == jnp.arange(N_E, dtype=assign.dtype)[None, :]
    cum = jnp.cumsum(onehot.astype(jnp.int32), axis=0)
    rank = jnp.sum(jnp.where(onehot, cum - 1, 0), axis=1)
    p = assign * CAP + rank

    return pl.pallas_call(
        _body,
        out_shape=jax.ShapeDtypeStruct((T, D), jnp.float32),
        in_specs=[
            pl.BlockSpec(memory_space=pltpu.VMEM),
            pl.BlockSpec(memory_space=pltpu.VMEM),
            pl.BlockSpec(memory_space=pl.ANY),
            pl.BlockSpec(memory_space=pl.ANY),
        ],
        out_specs=pl.BlockSpec(memory_space=pltpu.VMEM),
        scratch_shapes=[
            pltpu.VMEM((N_E, CAP, D), jnp.bfloat16),
            pltpu.VMEM((E_LOCAL, N_Z, CAP, D), jnp.bfloat16),
            pltpu.VMEM(W1.shape[1:], jnp.float32),
            pltpu.VMEM(W2.shape[1:], jnp.float32),
            pltpu.SemaphoreType.DMA((E_LOCAL,)),
            pltpu.SemaphoreType.DMA((E_LOCAL, N_Z)),
            pltpu.SemaphoreType.DMA((E_LOCAL, N_Z)),
            pltpu.SemaphoreType.DMA((E_LOCAL, N_Z)),
            pltpu.SemaphoreType.DMA((E_LOCAL, N_Z)),
            pltpu.SemaphoreType.DMA,
            pltpu.SemaphoreType.DMA,
        ],
        compiler_params=pltpu.CompilerParams(
            collective_id=0, vmem_limit_bytes=100 * 1024 * 1024
        ),
    )(x, p.astype(jnp.int32), W1, W2)


# device time: 94459 ns/iter; 9.4555x vs baseline; 1.6020x over previous
import jax
import jax.numpy as jnp
from jax import lax
from jax.experimental import pallas as pl
from jax.experimental.pallas import tpu as pltpu

N_Z = 4
E_LOCAL = 4
N_E = N_Z * E_LOCAL
CAP = 56
CHUNK = E_LOCAL * CAP
TQ = 512
XY_PEERS = ((0, 1), (1, 0), (1, 1))


def _body(x_ref, p_ref, w1_hbm, w2_hbm, out_ref, xs_ref, recv_ref,
          w1buf, w2buf, qsend, qrecv, loc_sems, fwd_send, fwd_recv,
          bwd_send, bwd_recv, xy_send, xy_recv, wsem1, wsem2):
    my_x = lax.axis_index("x")
    my_y = lax.axis_index("y")
    my_z = lax.axis_index("z")
    D = x_ref.shape[1]

    barrier_sem = pltpu.get_barrier_semaphore()
    for dz in range(1, N_Z):
        pl.semaphore_signal(
            barrier_sem, inc=1,
            device_id=(my_x, my_y, (my_z + dz) % N_Z),
            device_id_type=pl.DeviceIdType.MESH,
        )
    for ax, ay in XY_PEERS:
        pl.semaphore_signal(
            barrier_sem, inc=1,
            device_id=(my_x ^ ax, my_y ^ ay, my_z),
            device_id_type=pl.DeviceIdType.MESH,
        )
    pl.semaphore_wait(barrier_sem, N_Z - 1 + len(XY_PEERS))

    p_row = p_ref[...].reshape(1, TQ)
    xv = x_ref[...].astype(jnp.bfloat16)

    def gather_chunk(d):
        q = jax.lax.broadcasted_iota(jnp.int32, (CHUNK, TQ), 0) + d * CHUNK
        ohT = (q == p_row).astype(jnp.bfloat16)
        chunk = jax.lax.dot(ohT, xv, preferred_element_type=jnp.float32)
        xs_ref[pl.ds(E_LOCAL * d, E_LOCAL)] = (
            chunk.astype(jnp.bfloat16).reshape(E_LOCAL, CAP, D)
        )

    fwd = []
    for dz in range(1, N_Z):
        d = (my_z + dz) % N_Z
        gather_chunk(d)
        for j in range(E_LOCAL):
            rdma = pltpu.make_async_remote_copy(
                src_ref=xs_ref.at[pl.ds(E_LOCAL * d + j, 1)],
                dst_ref=recv_ref.at[j, pl.ds(dz, 1)],
                send_sem=fwd_send.at[j, dz],
                recv_sem=fwd_recv.at[j, dz],
                device_id=(my_x, my_y, d),
                device_id_type=pl.DeviceIdType.MESH,
            )
            rdma.start()
            fwd.append(rdma)
    gather_chunk(my_z)
    loc_fwd = []
    for j in range(E_LOCAL):
        cp = pltpu.make_async_copy(
            xs_ref.at[pl.ds(E_LOCAL * my_z + j, 1)],
            recv_ref.at[j, pl.ds(0, 1)],
            loc_sems.at[j],
        )
        cp.start()
        loc_fwd.append(cp)

    def load_w(hbm, j, buf, sem):
        cp = pltpu.make_async_copy(hbm.at[j], buf, sem)
        cp.start()
        return cp

    w1load = load_w(w1_hbm, 0, w1buf, wsem1)
    w2load = load_w(w2_hbm, 0, w2buf, wsem2)
    bwd = []
    for j in range(E_LOCAL):
        loc_fwd[j].wait()
        for b in range(1, N_Z):
            rcv = pltpu.make_async_remote_copy(
                src_ref=xs_ref.at[pl.ds(0, 1)],
                dst_ref=recv_ref.at[j, pl.ds(b, 1)],
                send_sem=fwd_send.at[j, b],
                recv_sem=fwd_recv.at[j, b],
                device_id=(my_x, my_y, my_z),
                device_id_type=pl.DeviceIdType.MESH,
            )
            rcv.wait_recv()
        a = recv_ref[j].reshape(N_Z * CAP, D).astype(jnp.float32)
        w1load.wait()
        h = jnp.maximum(
            jax.lax.dot(a, w1buf[...], preferred_element_type=jnp.float32), 0.0
        )
        if j + 1 < E_LOCAL:
            w1load = load_w(w1_hbm, j + 1, w1buf, wsem1)
        w2load.wait()
        o = jax.lax.dot(h, w2buf[...], preferred_element_type=jnp.float32)
        if j + 1 < E_LOCAL:
            w2load = load_w(w2_hbm, j + 1, w2buf, wsem2)
        recv_ref[j] = o.astype(jnp.bfloat16).reshape(N_Z, CAP, D)

        cp = pltpu.make_async_copy(
            recv_ref.at[j, pl.ds(0, 1)],
            xs_ref.at[pl.ds(E_LOCAL * my_z + j, 1)],
            loc_sems.at[j],
        )
        cp.start()
        loc_fwd[j] = cp
        for b in range(1, N_Z):
            s = (my_z - b) % N_Z
            rdma = pltpu.make_async_remote_copy(
                src_ref=recv_ref.at[j, pl.ds(b, 1)],
                dst_ref=xs_ref.at[pl.ds(E_LOCAL * my_z + j, 1)],
                send_sem=bwd_send.at[j, b],
                recv_sem=bwd_recv.at[j, b],
                device_id=(my_x, my_y, s),
                device_id_type=pl.DeviceIdType.MESH,
            )
            rdma.start()
            bwd.append(rdma)

    p_col = p_ref[...].reshape(TQ, 1)

    def scatter_chunk(d, acc):
        q = jax.lax.broadcasted_iota(jnp.int32, (TQ, CHUNK), 1) + d * CHUNK
        oh = (q == p_col).astype(jnp.bfloat16)
        res = xs_ref[pl.ds(E_LOCAL * d, E_LOCAL)].reshape(CHUNK, D)
        contrib = jax.lax.dot(oh, res, preferred_element_type=jnp.float32)
        return contrib if acc is None else acc + contrib

    for j in range(E_LOCAL):
        loc_fwd[j].wait()
    acc = scatter_chunk(my_z, None)
    for f in range(1, N_Z):
        d = (my_z + f) % N_Z
        for j in range(E_LOCAL):
            rcv = pltpu.make_async_remote_copy(
                src_ref=recv_ref.at[j, pl.ds(0, 1)],
                dst_ref=xs_ref.at[pl.ds(E_LOCAL * d + j, 1)],
                send_sem=bwd_send.at[j, f],
                recv_sem=bwd_recv.at[j, f],
                device_id=(my_x, my_y, my_z),
                device_id_type=pl.DeviceIdType.MESH,
            )
            rcv.wait_recv()
        acc = scatter_chunk(d, acc)

    my_q = 2 * my_x + my_y
    out_ref[pl.ds(my_q * TQ, TQ), :] = acc
    qsend[0] = acc.astype(jnp.bfloat16)

    xy = []
    for t, (ax, ay) in enumerate(XY_PEERS):
        rdma = pltpu.make_async_remote_copy(
            src_ref=qsend.at[pl.ds(0, 1)],
            dst_ref=qrecv.at[pl.ds(t, 1)],
            send_sem=xy_send.at[t],
            recv_sem=xy_recv.at[t],
            device_id=(my_x ^ ax, my_y ^ ay, my_z),
            device_id_type=pl.DeviceIdType.MESH,
        )
        rdma.start()
        xy.append(rdma)
    for t, (ax, ay) in enumerate(XY_PEERS):
        xy[t].wait_recv()
        peer_q = 2 * (my_x ^ ax) + (my_y ^ ay)
        out_ref[pl.ds(peer_q * TQ, TQ), :] = qrecv[t].astype(jnp.float32)

    for rdma in fwd:
        rdma.wait_send()
    for rdma in bwd:
        rdma.wait_send()
    for rdma in xy:
        rdma.wait_send()


def kernel(x, assign, W1, W2):
    T, D = x.shape

    my_q = 2 * lax.axis_index("x") + lax.axis_index("y")
    x_q = lax.dynamic_slice(x, (my_q * TQ, 0), (TQ, D))
    a_q = lax.dynamic_slice(assign, (my_q * TQ,), (TQ,))

    onehot = a_q[:, None] == jnp.arange(N_E, dtype=a_q.dtype)[None, :]
    cum = jnp.cumsum(onehot.astype(jnp.int32), axis=0)
    rank = jnp.sum(jnp.where(onehot, cum - 1, 0), axis=1)
    p = a_q * CAP + rank

    return pl.pallas_call(
        _body,
        out_shape=jax.ShapeDtypeStruct((T, D), jnp.float32),
        in_specs=[
            pl.BlockSpec(memory_space=pltpu.VMEM),
            pl.BlockSpec(memory_space=pltpu.VMEM),
            pl.BlockSpec(memory_space=pl.ANY),
            pl.BlockSpec(memory_space=pl.ANY),
        ],
        out_specs=pl.BlockSpec(memory_space=pltpu.VMEM),
        scratch_shapes=[
            pltpu.VMEM((N_E, CAP, D), jnp.bfloat16),
            pltpu.VMEM((E_LOCAL, N_Z, CAP, D), jnp.bfloat16),
            pltpu.VMEM(W1.shape[1:], jnp.float32),
            pltpu.VMEM(W2.shape[1:], jnp.float32),
            pltpu.VMEM((1, TQ, D), jnp.bfloat16),
            pltpu.VMEM((len(XY_PEERS), TQ, D), jnp.bfloat16),
            pltpu.SemaphoreType.DMA((E_LOCAL,)),
            pltpu.SemaphoreType.DMA((E_LOCAL, N_Z)),
            pltpu.SemaphoreType.DMA((E_LOCAL, N_Z)),
            pltpu.SemaphoreType.DMA((E_LOCAL, N_Z)),
            pltpu.SemaphoreType.DMA((E_LOCAL, N_Z)),
            pltpu.SemaphoreType.DMA((len(XY_PEERS),)),
            pltpu.SemaphoreType.DMA((len(XY_PEERS),)),
            pltpu.SemaphoreType.DMA,
            pltpu.SemaphoreType.DMA,
        ],
        compiler_params=pltpu.CompilerParams(
            collective_id=0, vmem_limit_bytes=100 * 1024 * 1024
        ),
    )(x_q, p.astype(jnp.int32), W1, W2)
